# baseline (speedup 1.0000x reference)
import functools

import jax
import jax.numpy as jnp
from jax import lax
from jax.experimental import pallas as pl
from jax.experimental.pallas import tpu as pltpu
from jax.experimental.pallas import tpu_sc as plsc

D = 768
H = 1536
E = 8
K = 2
N = 2048
PROJ = 256
CAP = 2 * ((K * N) // E)
NB = 3
NG = NB * E
A = K * N
AT = NB * A
T = 256
TPG = CAP // T
TRASH = NG * CAP
R_TOT = NG * CAP + T
CB = 512


def _gate_route_body(x_ref, pw_ref, pb_ref, sim_ref, ls_ref, lab_ref,
                     ds_ref, dest_ref, w_ref, cnt_ref, oh_ref, idx_ref):
    b = pl.program_id(0)
    x = x_ref[...]
    proj = jnp.dot(x.astype(jnp.bfloat16),
                   pw_ref[0].astype(jnp.bfloat16),
                   preferred_element_type=jnp.float32) + pb_ref[0]
    proj = proj / (jnp.sqrt(jnp.sum(proj * proj, axis=-1, keepdims=True))
                   + 1e-12)
    sim = sim_ref[0]
    simn = sim / (jnp.sqrt(jnp.sum(sim * sim, axis=0, keepdims=True))
                  + 1e-12)
    scale = jnp.exp(jnp.minimum(ls_ref[b, 0], jnp.log(1.0 / 0.01)))
    logits = jnp.dot(proj.astype(jnp.bfloat16),
                     simn.astype(jnp.bfloat16),
                     preferred_element_type=jnp.float32) * scale
    m = jnp.max(logits, axis=-1, keepdims=True)
    eg = jnp.exp(logits - m)
    gates = eg / jnp.sum(eg, axis=-1, keepdims=True)

    lanes = lax.broadcasted_iota(jnp.int32, (N, E), 1)
    v1 = jnp.max(gates, axis=-1, keepdims=True)
    i1 = jnp.min(jnp.where(gates == v1, lanes, E), axis=-1, keepdims=True)
    masked = jnp.where(lanes == i1, -jnp.inf, gates)
    v2 = jnp.max(masked, axis=-1, keepdims=True)
    i2 = jnp.min(jnp.where(masked == v2, lanes, E), axis=-1, keepdims=True)
    s = v1 + v2 + 1e-12
    g1, g2 = v1 / s, v2 / s

    lab = lab_ref[...]
    ds = ds_ref[0, 0]
    bscale = jnp.where(b == 0, jnp.ones((N, 1), jnp.float32),
                       ds * (lab == (b - 1)).astype(jnp.float32))
    w = jnp.concatenate([g1 * bscale, g2 * bscale], axis=0)
    idx = jnp.concatenate([i1, i2], axis=0)
    w_ref[0] = w
    idx_ref[...] = idx
    oh_ref[...] = (idx == lax.broadcasted_iota(jnp.int32, (A, E), 1)
                   ).astype(jnp.float32)
    r_i = lax.broadcasted_iota(jnp.int32, (CB, CB), 0)
    c_i = lax.broadcasted_iota(jnp.int32, (CB, CB), 1)
    tri = (c_i <= r_i).astype(jnp.float32)

    def blk(i, carry):
        base_pos, base_act = carry
        ohb = oh_ref[pl.ds(i * CB, CB), :]
        idx_b = idx_ref[pl.ds(i * CB, CB), :]
        w_b = w_ref[0, pl.ds(i * CB, CB), :]
        incl = jnp.dot(tri, ohb, preferred_element_type=jnp.float32,
                       precision=lax.Precision.HIGHEST)
        pos = jnp.sum((incl + base_pos - 1.0) * ohb, axis=-1,
                      keepdims=True)
        keep = pos < float(CAP)
        active = jnp.logical_and(keep, w_b != 0.0)
        aohb = ohb * active.astype(jnp.float32)
        incl_a = jnp.dot(tri, aohb, preferred_element_type=jnp.float32,
                         precision=lax.Precision.HIGHEST)
        slot = jnp.sum((incl_a + base_act - 1.0) * ohb, axis=-1,
                       keepdims=True).astype(jnp.int32)
        dest = jnp.where(active, idx_b * CAP + slot + b * (E * CAP),
                         -1)
        dest_ref[0, pl.ds(i * CB, CB), :] = dest
        return (base_pos + jnp.sum(ohb, axis=0, keepdims=True),
                base_act + jnp.sum(aohb, axis=0, keepdims=True))

    zero8 = jnp.zeros((1, E), jnp.float32)
    _, base_act = lax.fori_loop(0, A // CB, blk, (zero8, zero8))
    cnt_ref[0] = base_act.astype(jnp.int32)


def _gate_route(x, pw_all, pb_all, sim_all, ls_all, lab, ds):
    return pl.pallas_call(
        _gate_route_body,
        grid=(NB,),
        in_specs=[
            pl.BlockSpec((N, D), lambda b: (0, 0)),
            pl.BlockSpec((1, D, PROJ), lambda b: (b, 0, 0)),
            pl.BlockSpec((1, 1, PROJ), lambda b: (b, 0, 0)),
            pl.BlockSpec((1, PROJ, E), lambda b: (b, 0, 0)),
            pl.BlockSpec((NB, 1), lambda b: (0, 0),
                         memory_space=pltpu.SMEM),
            pl.BlockSpec((N, 1), lambda b: (0, 0)),
            pl.BlockSpec((1, 1), lambda b: (0, 0),
                         memory_space=pltpu.SMEM),
        ],
        out_specs=[
            pl.BlockSpec((1, A, 1), lambda b: (b, 0, 0)),
            pl.BlockSpec((1, A, 1), lambda b: (b, 0, 0)),
            pl.BlockSpec((1, 1, E), lambda b: (b, 0, 0)),
        ],
        out_shape=[
            jax.ShapeDtypeStruct((NB, A, 1), jnp.int32),
            jax.ShapeDtypeStruct((NB, A, 1), jnp.float32),
            jax.ShapeDtypeStruct((NB, 1, E), jnp.int32),
        ],
        scratch_shapes=[
            pltpu.VMEM((A, E), jnp.float32),
            pltpu.VMEM((A, 1), jnp.int32),
        ],
    )(x, pw_all, pb_all, sim_all, ls_all, lab, ds)


_SC_INFO = plsc.get_sparse_core_info()
_NTILES = _SC_INFO.num_cores * _SC_INFO.num_subcores
_DISP_C = 64
_PER_TILE = AT // _NTILES


def _dispatch_body(x_hbm, tok_hbm, dest_hbm, w_hbm, xe_hbm, wslot_hbm,
                   tok_v, dsel_v, w_v, rows_v, sem):
    wid = lax.axis_index("s") * _SC_INFO.num_cores + lax.axis_index("c")
    base = wid * _PER_TILE

    def chunk(ci, _):
        off = base + ci * _DISP_C
        pltpu.sync_copy(tok_hbm.at[pl.ds(off, _DISP_C)], tok_v)
        pltpu.sync_copy(dest_hbm.at[pl.ds(off, _DISP_C)], dsel_v)
        pltpu.sync_copy(w_hbm.at[pl.ds(off, _DISP_C)], w_v)

        def fix(c, _):
            d = dsel_v[pl.ds(c * 16, 16)]
            dsel_v[pl.ds(c * 16, 16)] = jnp.where(d < 0, TRASH, d)
            return 0
        lax.fori_loop(0, _DISP_C // 16, fix, 0)
        cp = pltpu.make_async_copy(x_hbm.at[tok_v], rows_v, sem)
        cp.start()
        cp.wait()
        pltpu.sync_copy(rows_v, xe_hbm.at[dsel_v])
        pltpu.sync_copy(w_v, wslot_hbm.at[dsel_v])
        return 0

    lax.fori_loop(0, _PER_TILE // _DISP_C, chunk, 0)


def _dispatch(x, tok, dest, w):
    kfn = pl.kernel(
        _dispatch_body,
        out_type=[jax.ShapeDtypeStruct((R_TOT, D), jnp.float32),
                  jax.ShapeDtypeStruct((R_TOT,), jnp.float32)],
        mesh=plsc.VectorSubcoreMesh(core_axis_name="c",
                                    subcore_axis_name="s"),
        scratch_types=[
            pltpu.VMEM((_DISP_C,), jnp.int32),
            pltpu.VMEM((_DISP_C,), jnp.int32),
            pltpu.VMEM((_DISP_C,), jnp.float32),
            pltpu.VMEM((_DISP_C, D), jnp.float32),
            pltpu.SemaphoreType.DMA,
        ],
    )
    return kfn(x, tok, dest, w)


def _ffn_body(cnt_ref, xe_ref, ws_ref, w1_ref, b1_ref, w2_ref, b2_ref,
              ye_ref):
    g = pl.program_id(0)
    j = pl.program_id(1)
    is_trash = g == NG

    @pl.when(jnp.logical_and(j * T < cnt_ref[g], jnp.logical_not(is_trash)))
    def _():
        xb = xe_ref[...].astype(jnp.bfloat16)
        h = jnp.dot(xb, w1_ref[0], preferred_element_type=jnp.float32)
        h = jnp.maximum(h + b1_ref[0], 0.0).astype(jnp.bfloat16)
        y = jnp.dot(h, w2_ref[0], preferred_element_type=jnp.float32)
        ye_ref[...] = (y + b2_ref[0]) * ws_ref[...]

    @pl.when(jnp.logical_and(j == 0, is_trash))
    def _():
        ye_ref[...] = jnp.zeros((T, D), jnp.float32)


def _ffn(counts, xe, wslot, w1_all, b1_all, w2_all, b2_all):
    def tile_idx(g, j, cnt):
        ntile = jnp.maximum((cnt[g] + (T - 1)) // T - 1, 0)
        return g * TPG + jnp.minimum(j, ntile)

    grid_spec = pltpu.PrefetchScalarGridSpec(
        num_scalar_prefetch=1,
        grid=(NG + 1, TPG),
        in_specs=[
            pl.BlockSpec((T, D), lambda g, j, cnt: (tile_idx(g, j, cnt), 0)),
            pl.BlockSpec((T, 1), lambda g, j, cnt: (tile_idx(g, j, cnt), 0)),
            pl.BlockSpec((1, D, H), lambda g, j, cnt: (jnp.minimum(g, NG - 1), 0, 0)),
            pl.BlockSpec((1, 1, H), lambda g, j, cnt: (jnp.minimum(g, NG - 1), 0, 0)),
            pl.BlockSpec((1, H, D), lambda g, j, cnt: (jnp.minimum(g, NG - 1), 0, 0)),
            pl.BlockSpec((1, 1, D), lambda g, j, cnt: (jnp.minimum(g, NG - 1), 0, 0)),
        ],
        out_specs=pl.BlockSpec((T, D),
                               lambda g, j, cnt: (tile_idx(g, j, cnt), 0)),
    )
    return pl.pallas_call(
        _ffn_body,
        grid_spec=grid_spec,
        out_shape=jax.ShapeDtypeStruct((R_TOT, D), jnp.float32),
    )(counts, xe, wslot, w1_all, b1_all, w2_all, b2_all)


_TOK_PT = N // _NTILES
_NJ = NB * K


def _combine_body(ye_hbm, dest_hbm, y_hbm,
                  dsel_v, rows_v, acc_v, sem):
    wid = lax.axis_index("s") * _SC_INFO.num_cores + lax.axis_index("c")
    base = wid * _TOK_PT

    def per_j(j, _):
        off = j * N + base
        pltpu.sync_copy(dest_hbm.at[pl.ds(off, _TOK_PT)], dsel_v)

        def fix(c, _):
            d = dsel_v[pl.ds(c * 16, 16)]
            dsel_v[pl.ds(c * 16, 16)] = jnp.where(d < 0, TRASH, d)
            return 0
        lax.fori_loop(0, _TOK_PT // 16, fix, 0)
        cp = pltpu.make_async_copy(ye_hbm.at[dsel_v], rows_v, sem)
        cp.start()
        cp.wait()

        def per_row(r, _):
            def per_col(c, _):
                contrib = rows_v[r, pl.ds(c * 16, 16)]
                prev = jnp.where(j == 0, jnp.zeros((16,), jnp.float32),
                                 acc_v[r, pl.ds(c * 16, 16)])
                acc_v[r, pl.ds(c * 16, 16)] = prev + contrib
                return 0
            lax.fori_loop(0, D // 16, per_col, 0)
            return 0
        lax.fori_loop(0, _TOK_PT, per_row, 0)
        return 0

    lax.fori_loop(0, _NJ, per_j, 0)
    pltpu.sync_copy(acc_v, y_hbm.at[pl.ds(base, _TOK_PT)])


def _combine(ye, dest):
    kfn = pl.kernel(
        _combine_body,
        out_type=jax.ShapeDtypeStruct((N, D), jnp.float32),
        mesh=plsc.VectorSubcoreMesh(core_axis_name="c",
                                    subcore_axis_name="s"),
        scratch_types=[
            pltpu.VMEM((_TOK_PT,), jnp.int32),
            pltpu.VMEM((_TOK_PT, D), jnp.float32),
            pltpu.VMEM((_TOK_PT, D), jnp.float32),
            pltpu.SemaphoreType.DMA,
        ],
    )
    return kfn(ye, dest)


def kernel(x, domain_label, domain_scale, s_pw, s_pb, s_sim, s_ls, s_w1,
           s_b1, s_w2, s_b2, d_pw, d_pb, d_sim, d_ls, d_w1, d_b1, d_w2,
           d_b2, n_pw, n_pb, n_sim, n_ls, n_w1, n_b1, n_w2, n_b2):
    pw_all = jnp.stack([s_pw, d_pw, n_pw])
    pb_all = jnp.stack([s_pb, d_pb, n_pb]).reshape(NB, 1, PROJ)
    sim_all = jnp.stack([s_sim, d_sim, n_sim])
    ls_all = jnp.stack([s_ls, d_ls, n_ls]).reshape(NB, 1)
    lab = domain_label.astype(jnp.int32).reshape(N, 1)
    ds = domain_scale.reshape(1, 1)

    dest3, w3, cnt3 = _gate_route(x, pw_all, pb_all, sim_all, ls_all,
                                  lab, ds)
    dest = dest3.reshape(AT)
    w = w3.reshape(AT)
    counts = jnp.concatenate(
        [cnt3.reshape(NG), jnp.ones((1,), jnp.int32)])

    tok = jnp.tile(jnp.arange(N, dtype=jnp.int32), NB * K)
    xe, wslot = _dispatch(x, tok, dest, w)
    wslot = wslot.reshape(R_TOT, 1)

    w1_all = jnp.stack([s_w1, d_w1, n_w1]).reshape(NG, D, H
                                                   ).astype(jnp.bfloat16)
    b1_all = jnp.stack([s_b1, d_b1, n_b1]).reshape(NG, 1, H)
    w2_all = jnp.stack([s_w2, d_w2, n_w2]).reshape(NG, H, D
                                                   ).astype(jnp.bfloat16)
    b2_all = jnp.stack([s_b2, d_b2, n_b2]).reshape(NG, 1, D)
    ye = _ffn(counts, xe, wslot, w1_all, b1_all, w2_all, b2_all)

    return _combine(ye, dest)


# speedup vs baseline: 2.0684x; 2.0684x over previous
import jax
import jax.numpy as jnp
from jax import lax
from jax.experimental import pallas as pl
from jax.experimental.pallas import tpu as pltpu
from jax.experimental.pallas import tpu_sc as plsc

D = 768
H = 1536
E = 8
K = 2
N = 2048
PROJ = 256
CAP = 2 * ((K * N) // E)
NB = 3
A = K * N
AT = NB * A
T = 256
TPG = CAP // T
TRASH = E * CAP
R_B = E * CAP + 8
CB = 512
L = 16


def _gate_route_body(x_ref, pw_ref, pb_ref, sim_ref, ls_ref, lab_ref,
                     ds_ref, dest_ref, w_ref, cnt_ref, oh_ref, idx_ref,
                     wv_ref):
    b = pl.program_id(0)
    x = x_ref[...]
    proj = jnp.dot(x.astype(jnp.bfloat16),
                   pw_ref[0].astype(jnp.bfloat16),
                   preferred_element_type=jnp.float32) + pb_ref[0]
    proj = proj / (jnp.sqrt(jnp.sum(proj * proj, axis=-1, keepdims=True))
                   + 1e-12)
    sim = sim_ref[0]
    simn = sim / (jnp.sqrt(jnp.sum(sim * sim, axis=0, keepdims=True))
                  + 1e-12)
    scale = jnp.exp(jnp.minimum(ls_ref[b, 0], jnp.log(1.0 / 0.01)))
    logits = jnp.dot(proj.astype(jnp.bfloat16),
                     simn.astype(jnp.bfloat16),
                     preferred_element_type=jnp.float32) * scale
    m = jnp.max(logits, axis=-1, keepdims=True)
    eg = jnp.exp(logits - m)
    gates = eg / jnp.sum(eg, axis=-1, keepdims=True)

    lanes = lax.broadcasted_iota(jnp.int32, (N, E), 1)
    v1 = jnp.max(gates, axis=-1, keepdims=True)
    i1 = jnp.min(jnp.where(gates == v1, lanes, E), axis=-1, keepdims=True)
    masked = jnp.where(lanes == i1, -jnp.inf, gates)
    v2 = jnp.max(masked, axis=-1, keepdims=True)
    i2 = jnp.min(jnp.where(masked == v2, lanes, E), axis=-1, keepdims=True)
    s = v1 + v2 + 1e-12
    g1, g2 = v1 / s, v2 / s

    lab = lab_ref[...]
    ds = ds_ref[0, 0]
    bscale = jnp.where(b == 0, jnp.ones((N, 1), jnp.float32),
                       ds * (lab == (b - 1)).astype(jnp.float32))
    wv_ref[...] = jnp.concatenate([g1 * bscale, g2 * bscale], axis=0)
    idx = jnp.concatenate([i1, i2], axis=0)
    idx_ref[...] = idx
    oh_ref[...] = (idx == lax.broadcasted_iota(jnp.int32, (A, E), 1)
                   ).astype(jnp.float32)
    r_i = lax.broadcasted_iota(jnp.int32, (CB, CB), 0)
    c_i = lax.broadcasted_iota(jnp.int32, (CB, CB), 1)
    tri = (c_i <= r_i).astype(jnp.float32)

    def blk(i, carry):
        base_pos, base_act = carry
        ohb = oh_ref[pl.ds(i * CB, CB), :]
        idx_b = idx_ref[pl.ds(i * CB, CB), :]
        w_b = wv_ref[pl.ds(i * CB, CB), :]
        incl = jnp.dot(tri, ohb, preferred_element_type=jnp.float32,
                       precision=lax.Precision.HIGHEST)
        pos = jnp.sum((incl + base_pos - 1.0) * ohb, axis=-1,
                      keepdims=True)
        keep = pos < float(CAP)
        active = jnp.logical_and(keep, w_b != 0.0)
        aohb = ohb * active.astype(jnp.float32)
        incl_a = jnp.dot(tri, aohb, preferred_element_type=jnp.float32,
                         precision=lax.Precision.HIGHEST)
        slot = jnp.sum((incl_a + base_act - 1.0) * ohb, axis=-1,
                       keepdims=True).astype(jnp.int32)
        dest = jnp.where(active, idx_b * CAP + slot, TRASH)
        dest_ref[0, pl.ds(i * CB, CB), :] = dest
        wm = jnp.where(active, w_b, 0.0)
        w_ref[0, pl.ds(i * CB, CB), :] = jnp.broadcast_to(wm, (CB, L))
        return (base_pos + jnp.sum(ohb, axis=0, keepdims=True),
                base_act + jnp.sum(aohb, axis=0, keepdims=True))

    zero8 = jnp.zeros((1, E), jnp.float32)
    _, base_act = lax.fori_loop(0, A // CB, blk, (zero8, zero8))
    cnt_ref[0] = base_act.astype(jnp.int32)


def _gate_route(x, pw_all, pb_all, sim_all, ls_all, lab, ds):
    return pl.pallas_call(
        _gate_route_body,
        grid=(NB,),
        in_specs=[
            pl.BlockSpec((N, D), lambda b: (0, 0)),
            pl.BlockSpec((1, D, PROJ), lambda b: (b, 0, 0)),
            pl.BlockSpec((1, 1, PROJ), lambda b: (b, 0, 0)),
            pl.BlockSpec((1, PROJ, E), lambda b: (b, 0, 0)),
            pl.BlockSpec((NB, 1), lambda b: (0, 0),
                         memory_space=pltpu.SMEM),
            pl.BlockSpec((N, 1), lambda b: (0, 0)),
            pl.BlockSpec((1, 1), lambda b: (0, 0),
                         memory_space=pltpu.SMEM),
        ],
        out_specs=[
            pl.BlockSpec((1, A, 1), lambda b: (b, 0, 0)),
            pl.BlockSpec((1, A, L), lambda b: (b, 0, 0)),
            pl.BlockSpec((1, 1, E), lambda b: (b, 0, 0)),
        ],
        out_shape=[
            jax.ShapeDtypeStruct((NB, A, 1), jnp.int32),
            jax.ShapeDtypeStruct((NB, A, L), jnp.float32),
            jax.ShapeDtypeStruct((NB, 1, E), jnp.int32),
        ],
        scratch_shapes=[
            pltpu.VMEM((A, E), jnp.float32),
            pltpu.VMEM((A, 1), jnp.int32),
            pltpu.VMEM((A, 1), jnp.float32),
        ],
    )(x, pw_all, pb_all, sim_all, ls_all, lab, ds)


_SC_INFO = plsc.get_sparse_core_info()
_NTILES = _SC_INFO.num_cores * _SC_INFO.num_subcores
_DISP_C = 64
_PER_TILE = A // _NTILES
_NCHUNK = _PER_TILE // _DISP_C


def _dispatch_body(x_hbm, tok_hbm, dest_hbm, xe_hbm,
                   tok0, tok1, dst0, dst1, rows0, rows1,
                   sg0, sg1, ss0, ss1):
    wid = lax.axis_index("s") * _SC_INFO.num_cores + lax.axis_index("c")
    base = wid * _PER_TILE
    toks = (tok0, tok1)
    dsts = (dst0, dst1)
    rows = (rows0, rows1)
    sgs = (sg0, sg1)
    sss = (ss0, ss1)

    def stage(ci):
        off = base + ci * _DISP_C
        p = ci % 2
        pltpu.sync_copy(tok_hbm.at[pl.ds(off, _DISP_C)], toks[p])
        pltpu.sync_copy(dest_hbm.at[pl.ds(off, _DISP_C)], dsts[p])
        pltpu.make_async_copy(x_hbm.at[toks[p]], rows[p], sgs[p]).start()

    stage(0)
    for ci in range(_NCHUNK):
        p = ci % 2
        if ci + 1 < _NCHUNK:
            if ci >= 1:
                pltpu.make_async_copy(rows[1 - p], xe_hbm.at[dsts[1 - p]],
                                      sss[1 - p]).wait()
            stage(ci + 1)
        pltpu.make_async_copy(x_hbm.at[toks[p]], rows[p], sgs[p]).wait()
        pltpu.make_async_copy(rows[p], xe_hbm.at[dsts[p]], sss[p]).start()
    for ci in range(max(_NCHUNK - 2, 0), _NCHUNK):
        p = ci % 2
        pltpu.make_async_copy(rows[p], xe_hbm.at[dsts[p]], sss[p]).wait()


def _dispatch(x, tok, dest):
    kfn = pl.kernel(
        _dispatch_body,
        out_type=jax.ShapeDtypeStruct((R_B, D), jnp.float32),
        mesh=plsc.VectorSubcoreMesh(core_axis_name="c",
                                    subcore_axis_name="s"),
        scratch_types=[
            pltpu.VMEM((_DISP_C,), jnp.int32),
            pltpu.VMEM((_DISP_C,), jnp.int32),
            pltpu.VMEM((_DISP_C,), jnp.int32),
            pltpu.VMEM((_DISP_C,), jnp.int32),
            pltpu.VMEM((_DISP_C, D), jnp.float32),
            pltpu.VMEM((_DISP_C, D), jnp.float32),
            pltpu.SemaphoreType.DMA,
            pltpu.SemaphoreType.DMA,
            pltpu.SemaphoreType.DMA,
            pltpu.SemaphoreType.DMA,
        ],
    )
    return kfn(x, tok, dest)


def _ffn_body(cnt_ref, xe_ref, w1_ref, b1_ref, w2_ref, b2_ref, ye_ref,
              w1bf_ref, w2bf_ref):
    g = pl.program_id(0)
    j = pl.program_id(1)

    @pl.when(j == 0)
    def _():
        w1bf_ref[...] = w1_ref[0].astype(jnp.bfloat16)
        w2bf_ref[...] = w2_ref[0].astype(jnp.bfloat16)

    @pl.when(j * T < cnt_ref[g])
    def _():
        xb = xe_ref[...].astype(jnp.bfloat16)
        h = jnp.dot(xb, w1bf_ref[...], preferred_element_type=jnp.float32)
        h = jnp.maximum(h + b1_ref[0], 0.0).astype(jnp.bfloat16)
        y = jnp.dot(h, w2bf_ref[...], preferred_element_type=jnp.float32)
        ye_ref[...] = y + b2_ref[0]


def _ffn(counts, xe, w1, b1, w2, b2):
    def tile_idx(g, j, cnt):
        ntile = jnp.maximum((cnt[g] + (T - 1)) // T - 1, 0)
        return g * TPG + jnp.minimum(j, ntile)

    grid_spec = pltpu.PrefetchScalarGridSpec(
        num_scalar_prefetch=1,
        grid=(E, TPG),
        in_specs=[
            pl.BlockSpec((T, D), lambda g, j, cnt: (tile_idx(g, j, cnt), 0)),
            pl.BlockSpec((1, D, H), lambda g, j, cnt: (g, 0, 0)),
            pl.BlockSpec((1, 1, H), lambda g, j, cnt: (g, 0, 0)),
            pl.BlockSpec((1, H, D), lambda g, j, cnt: (g, 0, 0)),
            pl.BlockSpec((1, 1, D), lambda g, j, cnt: (g, 0, 0)),
        ],
        out_specs=pl.BlockSpec((T, D),
                               lambda g, j, cnt: (tile_idx(g, j, cnt), 0)),
        scratch_shapes=[
            pltpu.VMEM((D, H), jnp.bfloat16),
            pltpu.VMEM((H, D), jnp.bfloat16),
        ],
    )
    return pl.pallas_call(
        _ffn_body,
        grid_spec=grid_spec,
        out_shape=jax.ShapeDtypeStruct((R_B, D), jnp.float32),
    )(counts, xe, w1, b1, w2, b2)


_TOK_C = 32
_NJ = NB * K


def _combine_body(ye_s, ye_d, ye_n, dest_hbm, w_hbm, y_hbm,
                  d0, d1, wr0, wr1, rows0, rows1, acc_v, sg0, sg1):
    wid = lax.axis_index("s") * _SC_INFO.num_cores + lax.axis_index("c")
    yes = (ye_s, ye_s, ye_d, ye_d, ye_n, ye_n)
    ds_ = (d0, d1)
    wrs = (wr0, wr1)
    rows = (rows0, rows1)
    sgs = (sg0, sg1)

    for hh in range(2):
        base = wid * (2 * _TOK_C) + hh * _TOK_C

        def stage(j):
            p = j % 2
            off = j * N + base
            pltpu.sync_copy(dest_hbm.at[pl.ds(off, _TOK_C)], ds_[p])
            pltpu.sync_copy(w_hbm.at[pl.ds(off, _TOK_C), :], wrs[p])
            pltpu.make_async_copy(yes[j].at[ds_[p]], rows[p],
                                  sgs[p]).start()

        stage(0)
        for j in range(_NJ):
            p = j % 2
            pltpu.make_async_copy(yes[j].at[ds_[p]], rows[p],
                                  sgs[p]).wait()
            if j + 1 < _NJ:
                stage(j + 1)

            def per_row(r, _):
                wv = wrs[p][r, :]
                wz = wv != 0.0

                def per_col(c, _):
                    contrib = jnp.where(
                        wz, wv * rows[p][r, pl.ds(c * L, L)], 0.0)
                    if j == 0:
                        acc_v[r, pl.ds(c * L, L)] = contrib
                    else:
                        acc_v[r, pl.ds(c * L, L)] += contrib
                    return 0
                lax.fori_loop(0, D // L, per_col, 0)
                return 0
            lax.fori_loop(0, _TOK_C, per_row, 0)
        pltpu.sync_copy(acc_v, y_hbm.at[pl.ds(base, _TOK_C)])


def _combine(ye_s, ye_d, ye_n, dest, w):
    kfn = pl.kernel(
        _combine_body,
        out_type=jax.ShapeDtypeStruct((N, D), jnp.float32),
        mesh=plsc.VectorSubcoreMesh(core_axis_name="c",
                                    subcore_axis_name="s"),
        scratch_types=[
            pltpu.VMEM((_TOK_C,), jnp.int32),
            pltpu.VMEM((_TOK_C,), jnp.int32),
            pltpu.VMEM((_TOK_C, L), jnp.float32),
            pltpu.VMEM((_TOK_C, L), jnp.float32),
            pltpu.VMEM((_TOK_C, D), jnp.float32),
            pltpu.VMEM((_TOK_C, D), jnp.float32),
            pltpu.VMEM((_TOK_C, D), jnp.float32),
            pltpu.SemaphoreType.DMA,
            pltpu.SemaphoreType.DMA,
        ],
    )
    return kfn(ye_s, ye_d, ye_n, dest, w)


def kernel(x, domain_label, domain_scale, s_pw, s_pb, s_sim, s_ls, s_w1,
           s_b1, s_w2, s_b2, d_pw, d_pb, d_sim, d_ls, d_w1, d_b1, d_w2,
           d_b2, n_pw, n_pb, n_sim, n_ls, n_w1, n_b1, n_w2, n_b2):
    pw_all = jnp.stack([s_pw, d_pw, n_pw])
    pb_all = jnp.stack([s_pb, d_pb, n_pb]).reshape(NB, 1, PROJ)
    sim_all = jnp.stack([s_sim, d_sim, n_sim])
    ls_all = jnp.stack([s_ls, d_ls, n_ls]).reshape(NB, 1)
    lab = domain_label.astype(jnp.int32).reshape(N, 1)
    ds = domain_scale.reshape(1, 1)

    dest3, w3, cnt3 = _gate_route(x, pw_all, pb_all, sim_all, ls_all,
                                  lab, ds)
    tok = jnp.tile(jnp.arange(N, dtype=jnp.int32), K)

    yes = []
    packs = [(s_w1, s_b1, s_w2, s_b2), (d_w1, d_b1, d_w2, d_b2),
             (n_w1, n_b1, n_w2, n_b2)]
    for b in range(NB):
        w1, b1, w2, b2 = packs[b]
        xe = _dispatch(x, tok, dest3[b].reshape(A))
        ye = _ffn(cnt3[b].reshape(E), xe, w1, b1.reshape(E, 1, H),
                  w2, b2.reshape(E, 1, D))
        yes.append(ye)

    return _combine(yes[0], yes[1], yes[2],
                    dest3.reshape(AT), w3.reshape(AT, L))


# speedup vs baseline: 2.0697x; 1.0006x over previous
import jax
import jax.numpy as jnp
from jax import lax
from jax.experimental import pallas as pl
from jax.experimental.pallas import tpu as pltpu
from jax.experimental.pallas import tpu_sc as plsc

D = 768
H = 1536
E = 8
K = 2
N = 2048
PROJ = 256
CAP = 2 * ((K * N) // E)
NB = 3
A = K * N
AT = NB * A
T = 256
TPG = CAP // T
TRASH = E * CAP
R_B = E * CAP + 256
CB = 512
L = 16


def _gate_route_body(x_ref, pw_ref, pb_ref, sim_ref, ls_ref, lab_ref,
                     ds_ref, dest_ref, w_ref, cnt_ref, oh_ref, idx_ref,
                     wv_ref):
    b = pl.program_id(0)
    x = x_ref[...]
    proj = jnp.dot(x.astype(jnp.bfloat16),
                   pw_ref[0].astype(jnp.bfloat16),
                   preferred_element_type=jnp.float32) + pb_ref[0]
    proj = proj / (jnp.sqrt(jnp.sum(proj * proj, axis=-1, keepdims=True))
                   + 1e-12)
    sim = sim_ref[0]
    simn = sim / (jnp.sqrt(jnp.sum(sim * sim, axis=0, keepdims=True))
                  + 1e-12)
    scale = jnp.exp(jnp.minimum(ls_ref[b, 0], jnp.log(1.0 / 0.01)))
    logits = jnp.dot(proj.astype(jnp.bfloat16),
                     simn.astype(jnp.bfloat16),
                     preferred_element_type=jnp.float32) * scale
    m = jnp.max(logits, axis=-1, keepdims=True)
    eg = jnp.exp(logits - m)
    gates = eg / jnp.sum(eg, axis=-1, keepdims=True)

    lanes = lax.broadcasted_iota(jnp.int32, (N, E), 1)
    v1 = jnp.max(gates, axis=-1, keepdims=True)
    i1 = jnp.min(jnp.where(gates == v1, lanes, E), axis=-1, keepdims=True)
    masked = jnp.where(lanes == i1, -jnp.inf, gates)
    v2 = jnp.max(masked, axis=-1, keepdims=True)
    i2 = jnp.min(jnp.where(masked == v2, lanes, E), axis=-1, keepdims=True)
    s = v1 + v2 + 1e-12
    g1, g2 = v1 / s, v2 / s

    lab = lab_ref[...]
    ds = ds_ref[0, 0]
    bscale = jnp.where(b == 0, jnp.ones((N, 1), jnp.float32),
                       ds * (lab == (b - 1)).astype(jnp.float32))
    wv_ref[...] = jnp.concatenate([g1 * bscale, g2 * bscale], axis=0)
    idx = jnp.concatenate([i1, i2], axis=0)
    idx_ref[...] = idx
    oh_ref[...] = (idx == lax.broadcasted_iota(jnp.int32, (A, E), 1)
                   ).astype(jnp.float32)
    r_i = lax.broadcasted_iota(jnp.int32, (CB, CB), 0)
    c_i = lax.broadcasted_iota(jnp.int32, (CB, CB), 1)
    tri = (c_i <= r_i).astype(jnp.float32)

    def blk(i, carry):
        base_pos, base_act = carry
        ohb = oh_ref[pl.ds(i * CB, CB), :]
        idx_b = idx_ref[pl.ds(i * CB, CB), :]
        w_b = wv_ref[pl.ds(i * CB, CB), :]
        incl = jnp.dot(tri, ohb, preferred_element_type=jnp.float32,
                       precision=lax.Precision.HIGHEST)
        pos = jnp.sum((incl + base_pos - 1.0) * ohb, axis=-1,
                      keepdims=True)
        keep = pos < float(CAP)
        active = jnp.logical_and(keep, w_b != 0.0)
        aohb = ohb * active.astype(jnp.float32)
        incl_a = jnp.dot(tri, aohb, preferred_element_type=jnp.float32,
                         precision=lax.Precision.HIGHEST)
        slot = jnp.sum((incl_a + base_act - 1.0) * ohb, axis=-1,
                       keepdims=True).astype(jnp.int32)
        dest = jnp.where(active, idx_b * CAP + slot, TRASH)
        dest_ref[0, pl.ds(i * CB, CB), :] = dest
        wm = jnp.where(active, w_b, 0.0)
        w_ref[0, pl.ds(i * CB, CB), :] = jnp.broadcast_to(wm, (CB, L))
        return (base_pos + jnp.sum(ohb, axis=0, keepdims=True),
                base_act + jnp.sum(aohb, axis=0, keepdims=True))

    zero8 = jnp.zeros((1, E), jnp.float32)
    _, base_act = lax.fori_loop(0, A // CB, blk, (zero8, zero8))
    cnt_ref[0] = base_act.astype(jnp.int32)


def _gate_route(x, pw_all, pb_all, sim_all, ls_all, lab, ds):
    return pl.pallas_call(
        _gate_route_body,
        grid=(NB,),
        in_specs=[
            pl.BlockSpec((N, D), lambda b: (0, 0)),
            pl.BlockSpec((1, D, PROJ), lambda b: (b, 0, 0)),
            pl.BlockSpec((1, 1, PROJ), lambda b: (b, 0, 0)),
            pl.BlockSpec((1, PROJ, E), lambda b: (b, 0, 0)),
            pl.BlockSpec((NB, 1), lambda b: (0, 0),
                         memory_space=pltpu.SMEM),
            pl.BlockSpec((N, 1), lambda b: (0, 0)),
            pl.BlockSpec((1, 1), lambda b: (0, 0),
                         memory_space=pltpu.SMEM),
        ],
        out_specs=[
            pl.BlockSpec((1, A, 1), lambda b: (b, 0, 0)),
            pl.BlockSpec((1, A, L), lambda b: (b, 0, 0)),
            pl.BlockSpec((1, 1, E), lambda b: (b, 0, 0)),
        ],
        out_shape=[
            jax.ShapeDtypeStruct((NB, A, 1), jnp.int32),
            jax.ShapeDtypeStruct((NB, A, L), jnp.float32),
            jax.ShapeDtypeStruct((NB, 1, E), jnp.int32),
        ],
        scratch_shapes=[
            pltpu.VMEM((A, E), jnp.float32),
            pltpu.VMEM((A, 1), jnp.int32),
            pltpu.VMEM((A, 1), jnp.float32),
        ],
    )(x, pw_all, pb_all, sim_all, ls_all, lab, ds)


_SC_INFO = plsc.get_sparse_core_info()
_NTILES = _SC_INFO.num_cores * _SC_INFO.num_subcores
_DISP_C = 64
_PER_TILE = A // _NTILES
_NCHUNK = _PER_TILE // _DISP_C


def _dispatch_body(x_hbm, tok_hbm, dest_hbm, xe_hbm,
                   tok0, tok1, dst0, dst1, rows0, rows1,
                   sg0, sg1, ss0, ss1):
    wid = lax.axis_index("s") * _SC_INFO.num_cores + lax.axis_index("c")
    base = wid * _PER_TILE
    toks = (tok0, tok1)
    dsts = (dst0, dst1)
    rows = (rows0, rows1)
    sgs = (sg0, sg1)
    sss = (ss0, ss1)

    def stage(ci):
        off = base + ci * _DISP_C
        p = ci % 2
        pltpu.sync_copy(tok_hbm.at[pl.ds(off, _DISP_C)], toks[p])
        pltpu.sync_copy(dest_hbm.at[pl.ds(off, _DISP_C)], dsts[p])
        pltpu.make_async_copy(x_hbm.at[toks[p]], rows[p], sgs[p]).start()

    stage(0)
    for ci in range(_NCHUNK):
        p = ci % 2
        if ci + 1 < _NCHUNK:
            if ci >= 1:
                pltpu.make_async_copy(rows[1 - p], xe_hbm.at[dsts[1 - p]],
                                      sss[1 - p]).wait()
            stage(ci + 1)
        pltpu.make_async_copy(x_hbm.at[toks[p]], rows[p], sgs[p]).wait()
        pltpu.make_async_copy(rows[p], xe_hbm.at[dsts[p]], sss[p]).start()
    for ci in range(max(_NCHUNK - 2, 0), _NCHUNK):
        p = ci % 2
        pltpu.make_async_copy(rows[p], xe_hbm.at[dsts[p]], sss[p]).wait()


def _dispatch(x, tok, dest):
    kfn = pl.kernel(
        _dispatch_body,
        out_type=jax.ShapeDtypeStruct((R_B, D), jnp.float32),
        mesh=plsc.VectorSubcoreMesh(core_axis_name="c",
                                    subcore_axis_name="s"),
        scratch_types=[
            pltpu.VMEM((_DISP_C,), jnp.int32),
            pltpu.VMEM((_DISP_C,), jnp.int32),
            pltpu.VMEM((_DISP_C,), jnp.int32),
            pltpu.VMEM((_DISP_C,), jnp.int32),
            pltpu.VMEM((_DISP_C, D), jnp.float32),
            pltpu.VMEM((_DISP_C, D), jnp.float32),
            pltpu.SemaphoreType.DMA,
            pltpu.SemaphoreType.DMA,
            pltpu.SemaphoreType.DMA,
            pltpu.SemaphoreType.DMA,
        ],
    )
    return kfn(x, tok, dest)


def _ffn_body(cnt_ref, xe_ref, w1_ref, b1_ref, w2_ref, b2_ref, ye_ref,
              w1bf_ref, w2bf_ref):
    g = pl.program_id(0)
    j = pl.program_id(1)

    @pl.when(jnp.logical_and(j == 0, g < E))
    def _():
        w1bf_ref[...] = w1_ref[0].astype(jnp.bfloat16)
        w2bf_ref[...] = w2_ref[0].astype(jnp.bfloat16)

    @pl.when(jnp.logical_and(j * T < cnt_ref[g], g < E))
    def _():
        xb = xe_ref[...].astype(jnp.bfloat16)
        h = jnp.dot(xb, w1bf_ref[...], preferred_element_type=jnp.float32)
        h = jnp.maximum(h + b1_ref[0], 0.0).astype(jnp.bfloat16)
        y = jnp.dot(h, w2bf_ref[...], preferred_element_type=jnp.float32)
        ye_ref[...] = y + b2_ref[0]

    @pl.when(jnp.logical_and(j == 0, g == E))
    def _():
        ye_ref[...] = jnp.zeros((T, D), jnp.float32)


def _ffn(counts, xe, w1, b1, w2, b2):
    def tile_idx(g, j, cnt):
        ntile = jnp.maximum((cnt[g] + (T - 1)) // T - 1, 0)
        return g * TPG + jnp.minimum(j, ntile)

    grid_spec = pltpu.PrefetchScalarGridSpec(
        num_scalar_prefetch=1,
        grid=(E + 1, TPG),
        in_specs=[
            pl.BlockSpec((T, D), lambda g, j, cnt: (tile_idx(g, j, cnt), 0)),
            pl.BlockSpec((1, D, H),
                         lambda g, j, cnt: (jnp.minimum(g, E - 1), 0, 0)),
            pl.BlockSpec((1, 1, H),
                         lambda g, j, cnt: (jnp.minimum(g, E - 1), 0, 0)),
            pl.BlockSpec((1, H, D),
                         lambda g, j, cnt: (jnp.minimum(g, E - 1), 0, 0)),
            pl.BlockSpec((1, 1, D),
                         lambda g, j, cnt: (jnp.minimum(g, E - 1), 0, 0)),
        ],
        out_specs=pl.BlockSpec((T, D),
                               lambda g, j, cnt: (tile_idx(g, j, cnt), 0)),
        scratch_shapes=[
            pltpu.VMEM((D, H), jnp.bfloat16),
            pltpu.VMEM((H, D), jnp.bfloat16),
        ],
    )
    return pl.pallas_call(
        _ffn_body,
        grid_spec=grid_spec,
        out_shape=jax.ShapeDtypeStruct((R_B, D), jnp.float32),
    )(counts, xe, w1, b1, w2, b2)


_TOK_C = 32
_NJ = NB * K


def _combine_body(ye_s, ye_d, ye_n, dest_hbm, w_hbm, y_hbm,
                  d0, d1, wr0, wr1, rows0, rows1, acc_v, sg0, sg1):
    wid = lax.axis_index("s") * _SC_INFO.num_cores + lax.axis_index("c")
    yes = (ye_s, ye_s, ye_d, ye_d, ye_n, ye_n)
    ds_ = (d0, d1)
    wrs = (wr0, wr1)
    rows = (rows0, rows1)
    sgs = (sg0, sg1)

    for hh in range(2):
        base = wid * (2 * _TOK_C) + hh * _TOK_C

        def stage(j):
            p = j % 2
            off = j * N + base
            pltpu.sync_copy(dest_hbm.at[pl.ds(off, _TOK_C)], ds_[p])
            pltpu.sync_copy(w_hbm.at[pl.ds(off, _TOK_C), :], wrs[p])
            pltpu.make_async_copy(yes[j].at[ds_[p]], rows[p],
                                  sgs[p]).start()

        stage(0)
        for j in range(_NJ):
            p = j % 2
            pltpu.make_async_copy(yes[j].at[ds_[p]], rows[p],
                                  sgs[p]).wait()
            if j + 1 < _NJ:
                stage(j + 1)

            def per_row(r, _):
                wv = wrs[p][r, :]
                for c in range(D // L):
                    contrib = wv * rows[p][r, pl.ds(c * L, L)]
                    if j == 0:
                        acc_v[r, pl.ds(c * L, L)] = contrib
                    else:
                        acc_v[r, pl.ds(c * L, L)] += contrib
                return 0
            lax.fori_loop(0, _TOK_C, per_row, 0)
        pltpu.sync_copy(acc_v, y_hbm.at[pl.ds(base, _TOK_C)])


def _combine(ye_s, ye_d, ye_n, dest, w):
    kfn = pl.kernel(
        _combine_body,
        out_type=jax.ShapeDtypeStruct((N, D), jnp.float32),
        mesh=plsc.VectorSubcoreMesh(core_axis_name="c",
                                    subcore_axis_name="s"),
        scratch_types=[
            pltpu.VMEM((_TOK_C,), jnp.int32),
            pltpu.VMEM((_TOK_C,), jnp.int32),
            pltpu.VMEM((_TOK_C, L), jnp.float32),
            pltpu.VMEM((_TOK_C, L), jnp.float32),
            pltpu.VMEM((_TOK_C, D), jnp.float32),
            pltpu.VMEM((_TOK_C, D), jnp.float32),
            pltpu.VMEM((_TOK_C, D), jnp.float32),
            pltpu.SemaphoreType.DMA,
            pltpu.SemaphoreType.DMA,
        ],
    )
    return kfn(ye_s, ye_d, ye_n, dest, w)


def kernel(x, domain_label, domain_scale, s_pw, s_pb, s_sim, s_ls, s_w1,
           s_b1, s_w2, s_b2, d_pw, d_pb, d_sim, d_ls, d_w1, d_b1, d_w2,
           d_b2, n_pw, n_pb, n_sim, n_ls, n_w1, n_b1, n_w2, n_b2):
    pw_all = jnp.stack([s_pw, d_pw, n_pw])
    pb_all = jnp.stack([s_pb, d_pb, n_pb]).reshape(NB, 1, PROJ)
    sim_all = jnp.stack([s_sim, d_sim, n_sim])
    ls_all = jnp.stack([s_ls, d_ls, n_ls]).reshape(NB, 1)
    lab = domain_label.astype(jnp.int32).reshape(N, 1)
    ds = domain_scale.reshape(1, 1)

    dest3, w3, cnt3 = _gate_route(x, pw_all, pb_all, sim_all, ls_all,
                                  lab, ds)
    tok = jnp.tile(jnp.arange(N, dtype=jnp.int32), K)

    yes = []
    packs = [(s_w1, s_b1, s_w2, s_b2), (d_w1, d_b1, d_w2, d_b2),
             (n_w1, n_b1, n_w2, n_b2)]
    for b in range(NB):
        w1, b1, w2, b2 = packs[b]
        xe = _dispatch(x, tok, dest3[b].reshape(A))
        cnt9 = jnp.concatenate([cnt3[b].reshape(E),
                                jnp.ones((1,), jnp.int32)])
        ye = _ffn(cnt9, xe, w1, b1.reshape(E, 1, H),
                  w2, b2.reshape(E, 1, D))
        yes.append(ye)

    return _combine(yes[0], yes[1], yes[2],
                    dest3.reshape(AT), w3.reshape(AT, L))
